# baseline (speedup 1.0000x reference)
import functools

import jax
import jax.numpy as jnp
from jax.experimental import pallas as pl
from jax.experimental.pallas import tpu as pltpu

_C1, _H1, _W1 = 64, 32, 32
_C2, _H2, _W2 = 128, 16, 16
_IMGS_PER_STEP = 8


def _resize_matrix(out_size: int, in_size: int) -> jax.Array:
    scale = in_size / out_size
    src = (jnp.arange(out_size, dtype=jnp.float32) + 0.5) * scale - 0.5
    src = jnp.maximum(src, 0.0)
    i0 = jnp.minimum(jnp.floor(src).astype(jnp.int32), in_size - 1)
    i1 = jnp.minimum(i0 + 1, in_size - 1)
    frac = src - i0.astype(jnp.float32)
    rows = jnp.arange(out_size)
    m = jnp.zeros((out_size, in_size), jnp.float32)
    m = m.at[rows, i0].add(1.0 - frac)
    m = m.at[rows, i1].add(frac)
    return m


def _pool_matrix() -> jax.Array:
    rh = _resize_matrix(_H2, _H1)
    rw = _resize_matrix(_W2, _W1)
    return jnp.kron(rh[:8, :16], rw).T


def _body(x_ref, p_ref, w2_ref, b2_ref, o_ref, *, g):
    pooled = jnp.dot(x_ref[...], p_ref[...],
                     preferred_element_type=jnp.float32)
    b2 = b2_ref[...]
    w2 = w2_ref[...]
    for i in range(g):
        y = jnp.dot(w2, pooled[i * 128:(i + 1) * 128, :],
                    preferred_element_type=jnp.float32)
        o_ref[i * 256:(i + 1) * 256, :] = y + b2


@jax.jit
def kernel(x_nchw, weight, bias):
    n = x_nchw.shape[0]
    g = _IMGS_PER_STEP if n % _IMGS_PER_STEP == 0 else 1

    x2 = x_nchw.reshape(n * 128, 512)
    p = _pool_matrix()
    w2 = jnp.kron(weight.astype(jnp.float32), jnp.eye(2, dtype=jnp.float32))
    b2 = jnp.repeat(bias.astype(jnp.float32), 2).reshape(_C2 * 2, 1)

    out = pl.pallas_call(
        functools.partial(_body, g=g),
        out_shape=jax.ShapeDtypeStruct((n * 256, 128), x_nchw.dtype),
        grid_spec=pltpu.PrefetchScalarGridSpec(
            num_scalar_prefetch=0,
            grid=(n // g,),
            in_specs=[
                pl.BlockSpec((g * 128, 512), lambda i: (i, 0)),
                pl.BlockSpec((512, 128), lambda i: (0, 0)),
                pl.BlockSpec((256, 128), lambda i: (0, 0)),
                pl.BlockSpec((256, 1), lambda i: (0, 0)),
            ],
            out_specs=pl.BlockSpec((g * 256, 128), lambda i: (i, 0)),
        ),
        compiler_params=pltpu.CompilerParams(
            dimension_semantics=("parallel",),
            vmem_limit_bytes=64 << 20,
        ),
    )(x2, p, w2, b2)
    return out.reshape(n, _C2, _H2, _W2)


# speedup vs baseline: 3.0885x; 3.0885x over previous
import functools

import jax
import jax.numpy as jnp
from jax.experimental import pallas as pl
from jax.experimental.pallas import tpu as pltpu

_C1, _H1, _W1 = 64, 32, 32
_C2, _H2, _W2 = 128, 16, 16
_IMGS_PER_STEP = 16


def _resize_matrix(out_size: int, in_size: int) -> jax.Array:
    scale = in_size / out_size
    src = (jnp.arange(out_size, dtype=jnp.float32) + 0.5) * scale - 0.5
    src = jnp.maximum(src, 0.0)
    i0 = jnp.minimum(jnp.floor(src).astype(jnp.int32), in_size - 1)
    i1 = jnp.minimum(i0 + 1, in_size - 1)
    frac = src - i0.astype(jnp.float32)
    rows = jnp.arange(out_size)
    m = jnp.zeros((out_size, in_size), jnp.float32)
    m = m.at[rows, i0].add(1.0 - frac)
    m = m.at[rows, i1].add(frac)
    return m


def _body(x_ref, mt_ref, w_ref, b_ref, o_ref, *, g):
    x = x_ref[...].reshape(g * _C1, _H1 * _W1)
    pooled = jnp.dot(x, mt_ref[...],
                     preferred_element_type=jnp.float32)
    w = w_ref[...]
    b = b_ref[...]
    for i in range(g):
        y = jnp.dot(w, pooled[i * _C1:(i + 1) * _C1, :],
                    preferred_element_type=jnp.float32)
        o_ref[i, :, :] = y + b


@jax.jit
def kernel(x_nchw, weight, bias):
    n = x_nchw.shape[0]
    g = _IMGS_PER_STEP if n % _IMGS_PER_STEP == 0 else 1
    hw1, hw2 = _H1 * _W1, _H2 * _W2

    x3 = x_nchw.reshape(n, _C1, hw1)
    mt = jnp.kron(_resize_matrix(_H2, _H1), _resize_matrix(_W2, _W1)).T
    w = weight.astype(jnp.float32)
    b = bias.astype(jnp.float32).reshape(_C2, 1)

    out = pl.pallas_call(
        functools.partial(_body, g=g),
        out_shape=jax.ShapeDtypeStruct((n, _C2, hw2), x_nchw.dtype),
        grid_spec=pltpu.PrefetchScalarGridSpec(
            num_scalar_prefetch=0,
            grid=(n // g,),
            in_specs=[
                pl.BlockSpec((g, _C1, hw1), lambda i: (i, 0, 0)),
                pl.BlockSpec((hw1, hw2), lambda i: (0, 0)),
                pl.BlockSpec((_C2, _C1), lambda i: (0, 0)),
                pl.BlockSpec((_C2, 1), lambda i: (0, 0)),
            ],
            out_specs=pl.BlockSpec((g, _C2, hw2), lambda i: (i, 0, 0)),
        ),
        compiler_params=pltpu.CompilerParams(
            dimension_semantics=("parallel",),
            vmem_limit_bytes=64 << 20,
        ),
    )(x3, mt, w, b)
    return out.reshape(n, _C2, _H2, _W2)


# speedup vs baseline: 4.3623x; 1.4124x over previous
import functools

import jax
import jax.numpy as jnp
import numpy as np
from jax.experimental import pallas as pl
from jax.experimental.pallas import tpu as pltpu

_C1, _H1, _W1 = 64, 32, 32
_C2, _H2, _W2 = 128, 16, 16
_IMGS_PER_STEP = 16


def _resize_matrix(out_size: int, in_size: int) -> np.ndarray:
    scale = in_size / out_size
    src = (np.arange(out_size, dtype=np.float32) + 0.5) * scale - 0.5
    src = np.maximum(src, 0.0)
    i0 = np.minimum(np.floor(src).astype(np.int32), in_size - 1)
    i1 = np.minimum(i0 + 1, in_size - 1)
    frac = (src - i0.astype(np.float32)).astype(np.float32)
    rows = np.arange(out_size)
    m = np.zeros((out_size, in_size), np.float32)
    np.add.at(m, (rows, i0), 1.0 - frac)
    np.add.at(m, (rows, i1), frac)
    return m


_MT = np.kron(_resize_matrix(_H2, _H1), _resize_matrix(_W2, _W1)).T.copy()


def _body(x_ref, mt_ref, w_ref, b_ref, o_ref, *, g):
    x = x_ref[...].reshape(g * _C1, _H1 * _W1)
    pooled = jnp.dot(x, mt_ref[...],
                     preferred_element_type=jnp.float32)
    w = w_ref[...]
    b = b_ref[...]
    for i in range(g):
        y = jnp.dot(w, pooled[i * _C1:(i + 1) * _C1, :],
                    preferred_element_type=jnp.float32)
        o_ref[i, :, :] = y + b


@jax.jit
def kernel(x_nchw, weight, bias):
    n = x_nchw.shape[0]
    g = _IMGS_PER_STEP if n % _IMGS_PER_STEP == 0 else 1
    hw1, hw2 = _H1 * _W1, _H2 * _W2

    x3 = x_nchw.reshape(n, _C1, hw1)
    mt = jnp.asarray(_MT)
    w = weight.astype(jnp.float32)
    b = bias.astype(jnp.float32).reshape(_C2, 1)

    out = pl.pallas_call(
        functools.partial(_body, g=g),
        out_shape=jax.ShapeDtypeStruct((n, _C2, hw2), x_nchw.dtype),
        grid_spec=pltpu.PrefetchScalarGridSpec(
            num_scalar_prefetch=0,
            grid=(n // g,),
            in_specs=[
                pl.BlockSpec((g, _C1, hw1), lambda i: (i, 0, 0)),
                pl.BlockSpec((hw1, hw2), lambda i: (0, 0)),
                pl.BlockSpec((_C2, _C1), lambda i: (0, 0)),
                pl.BlockSpec((_C2, 1), lambda i: (0, 0)),
            ],
            out_specs=pl.BlockSpec((g, _C2, hw2), lambda i: (i, 0, 0)),
        ),
        compiler_params=pltpu.CompilerParams(
            dimension_semantics=("parallel",),
            vmem_limit_bytes=64 << 20,
        ),
    )(x3, mt, w, b)
    return out.reshape(n, _C2, _H2, _W2)
